# baseline (speedup 1.0000x reference)
import functools

import numpy as np

import jax
import jax.numpy as jnp
from jax import lax
from jax.experimental import pallas as pl
from jax.experimental.pallas import tpu as pltpu
from jax.experimental.pallas import tpu_sc as plsc

MAX_LEN = 200
VOCAB = 1000000
DIM = 64
BATCH = 4096

_INFO = plsc.get_sparse_core_info()
NC = _INFO.num_cores
NS = _INFO.num_subcores
LANES = _INFO.num_lanes
NW = NC * NS

CHUNK = 100
ROWS_PER_W = BATCH * MAX_LEN // NW
CHUNKS_PER_W = ROWS_PER_W // CHUNK
VECS_PER_ROW = DIM // LANES


def _pos_encoding() -> np.ndarray:
    pos = np.arange(MAX_LEN, dtype=np.float64)[:, None]
    i = np.arange(0, DIM, 2, dtype=np.float64)[None, :]
    angle = pos / (10000.0 ** (2.0 * i / DIM))
    enc = np.zeros((MAX_LEN, DIM), dtype=np.float64)
    enc[:, 0::2] = np.sin(angle)
    enc[:, 1::2] = np.cos(angle)
    return enc.astype(np.float32)


def _make_sc_kernel():
    mesh = plsc.VectorSubcoreMesh(core_axis_name="c", subcore_axis_name="s")

    @functools.partial(
        pl.kernel,
        mesh=mesh,
        compiler_params=pltpu.CompilerParams(use_tc_tiling_on_sc=False),
        out_type=jax.ShapeDtypeStruct(
            (BATCH * MAX_LEN // CHUNK, CHUNK, DIM), jnp.float32
        ),
        scratch_types=[
            pltpu.VMEM((CHUNKS_PER_W, CHUNK), jnp.int32),
            pltpu.VMEM((CHUNK, DIM), jnp.float32),
            pltpu.VMEM((MAX_LEN, DIM), jnp.float32),
            pltpu.SemaphoreType.DMA,
        ],
    )
    def k(idx_hbm, table_hbm, pe_hbm, out_hbm, idx_v, rows_v, pe_v, gsem):
        wid = lax.axis_index("s") * NC + lax.axis_index("c")
        pltpu.sync_copy(idx_hbm.at[wid], idx_v)
        pltpu.sync_copy(pe_hbm, pe_v)

        def pair_body(p, carry):
            for h in range(2):
                c = 2 * p + h
                pltpu.async_copy(table_hbm.at[idx_v.at[c]], rows_v, gsem).wait()

                def add_row(r, carry2):
                    for g in range(VECS_PER_ROW):
                        sl = pl.ds(g * LANES, LANES)
                        rows_v[r, sl] += pe_v[h * CHUNK + r, sl]
                    return carry2

                lax.fori_loop(0, CHUNK, add_row, 0, unroll=2)
                pltpu.sync_copy(rows_v, out_hbm.at[wid * CHUNKS_PER_W + c])
            return carry

        lax.fori_loop(0, CHUNKS_PER_W // 2, pair_body, 0)

    return k


_SC_KERNEL = _make_sc_kernel()


def kernel(batch, table):
    idx = batch.astype(jnp.int32).reshape(NW, CHUNKS_PER_W, CHUNK)
    pe = jnp.asarray(_pos_encoding())
    out = _SC_KERNEL(idx, table, pe)
    return out.reshape(BATCH, MAX_LEN, DIM)


# speedup vs baseline: 1.1980x; 1.1980x over previous
import functools

import numpy as np

import jax
import jax.numpy as jnp
from jax import lax
from jax.experimental import pallas as pl
from jax.experimental.pallas import tpu as pltpu
from jax.experimental.pallas import tpu_sc as plsc

MAX_LEN = 200
VOCAB = 1000000
DIM = 64
BATCH = 4096

_INFO = plsc.get_sparse_core_info()
NC = _INFO.num_cores
NS = _INFO.num_subcores
NW = NC * NS

CHUNK = 100
ROWS_PER_W = BATCH * MAX_LEN // NW
CHUNKS_PER_W = ROWS_PER_W // CHUNK
LAST = CHUNKS_PER_W - 1
NBUF = 4


def _pos_encoding() -> np.ndarray:
    pos = np.arange(MAX_LEN, dtype=np.float64)[:, None]
    i = np.arange(0, DIM, 2, dtype=np.float64)[None, :]
    angle = pos / (10000.0 ** (2.0 * i / DIM))
    enc = np.zeros((MAX_LEN, DIM), dtype=np.float64)
    enc[:, 0::2] = np.sin(angle)
    enc[:, 1::2] = np.cos(angle)
    return enc.astype(np.float32)


def _make_sc_kernel():
    mesh = plsc.VectorSubcoreMesh(core_axis_name="c", subcore_axis_name="s")

    @functools.partial(
        pl.kernel,
        mesh=mesh,
        compiler_params=pltpu.CompilerParams(use_tc_tiling_on_sc=False),
        out_type=jax.ShapeDtypeStruct(
            (BATCH * MAX_LEN // CHUNK, CHUNK, DIM), jnp.float32
        ),
        scratch_types=[
            pltpu.VMEM((CHUNKS_PER_W, CHUNK), jnp.int32),
            pltpu.VMEM((CHUNK, DIM), jnp.float32),
            pltpu.VMEM((CHUNK, DIM), jnp.float32),
            pltpu.VMEM((CHUNK, DIM), jnp.float32),
            pltpu.VMEM((CHUNK, DIM), jnp.float32),
            pltpu.SemaphoreType.DMA,
            pltpu.SemaphoreType.DMA,
            pltpu.SemaphoreType.DMA,
            pltpu.SemaphoreType.DMA,
        ],
    )
    def k(idx_hbm, table_hbm, pe_hbm, out_hbm,
          idx_v, ob0, ob1, ob2, ob3, s0, s1, s2, s3):
        obufs = (ob0, ob1, ob2, ob3)
        sems = (s0, s1, s2, s3)
        wid = lax.axis_index("s") * NC + lax.axis_index("c")
        base = wid * CHUNKS_PER_W
        pltpu.sync_copy(idx_hbm.at[wid], idx_v)

        def q_body(q, carry):
            for j in range(NBUF):
                c = NBUF * q + j

                @pl.when(c >= NBUF)
                def _():
                    pltpu.make_async_copy(
                        obufs[j], out_hbm.at[base + c - NBUF], sems[j]
                    ).wait()

                @pl.when(c <= LAST)
                def _():
                    pltpu.async_copy(pe_hbm.at[j % 2], obufs[j], sems[j])

                c1, j1 = c - 1, (j - 1) % NBUF

                @pl.when((c1 >= 0) & (c1 <= LAST))
                def _():
                    pltpu.make_async_copy(
                        pe_hbm.at[j1 % 2], obufs[j1], sems[j1]
                    ).wait()
                    pltpu.async_copy(
                        table_hbm.at[idx_v.at[c1]], obufs[j1], sems[j1], add=True
                    )

                c2, j2 = c - 2, (j - 2) % NBUF

                @pl.when((c2 >= 0) & (c2 <= LAST))
                def _():
                    pltpu.make_async_copy(
                        table_hbm.at[idx_v.at[c2]], obufs[j2], sems[j2]
                    ).wait()
                    pltpu.async_copy(obufs[j2], out_hbm.at[base + c2], sems[j2])

            return carry

        lax.fori_loop(0, CHUNKS_PER_W // NBUF + 1, q_body, 0)

    return k


_SC_KERNEL = _make_sc_kernel()


def kernel(batch, table):
    idx = batch.astype(jnp.int32).reshape(NW, CHUNKS_PER_W, CHUNK)
    pe = jnp.asarray(_pos_encoding()).reshape(2, CHUNK, DIM)
    out = _SC_KERNEL(idx, table, pe)
    return out.reshape(BATCH, MAX_LEN, DIM)
